# baseline (speedup 1.0000x reference)
import functools

import jax
import jax.numpy as jnp
from jax import lax
from jax.experimental import pallas as pl
from jax.experimental.pallas import tpu as pltpu
from jax.experimental.pallas import tpu_sc as plsc

N_NODES = 10000
N_EDGES = 320000
D = 128

NC = 2
NS = 16
NW = NC * NS

EPW = N_EDGES // NW
CHUNK = 128
NCHUNK = 81
EPWP = NCHUNK * CHUNK
NPAIR = NCHUNK // 2
PAD_DST = N_NODES

NPD = 10240
RPSD = NPD // NS
NPA = 10112
RPSA = NPA // NS

DEG_AHEAD = 8

ROW_BLK = 1000
N_BLKS = N_NODES // ROW_BLK

_sc_mesh = plsc.VectorSubcoreMesh(core_axis_name="c", subcore_axis_name="s")


@functools.partial(
    pl.kernel,
    out_type=jax.ShapeDtypeStruct((NC, NPD), jnp.float32),
    mesh=_sc_mesh,
    scratch_types=[
        pltpu.VMEM((NCHUNK, 1, CHUNK), jnp.int32),
        pltpu.VMEM((CHUNK,), jnp.float32),
        pltpu.VMEM_SHARED((NPD,), jnp.float32),
        pltpu.SemaphoreType.DMA,
    ],
)
def _deg_kernel(dst_hbm, ones_hbm, zeros_hbm, deg_out, idx_v, ones_v, deg_sh,
                sem):
    cid = lax.axis_index("c")
    sid = lax.axis_index("s")
    wid = sid * NC + cid
    pltpu.sync_copy(zeros_hbm, deg_sh.at[pl.ds(sid * RPSD, RPSD)])
    pltpu.sync_copy(ones_hbm, ones_v)
    pltpu.sync_copy(dst_hbm.at[wid], idx_v)
    plsc.subcore_barrier()

    for i in range(DEG_AHEAD):
        pltpu.async_copy(ones_v, deg_sh.at[idx_v.at[i, 0]], sem, add=True)

    def body(i, carry):
        pltpu.make_async_copy(ones_v, deg_sh.at[idx_v.at[0, 0]], sem).wait()
        pltpu.async_copy(ones_v, deg_sh.at[idx_v.at[i + DEG_AHEAD, 0]], sem,
                         add=True)
        return carry

    lax.fori_loop(0, NCHUNK - DEG_AHEAD, body, 0)
    for _ in range(DEG_AHEAD):
        pltpu.make_async_copy(ones_v, deg_sh.at[idx_v.at[0, 0]], sem).wait()
    plsc.subcore_barrier()
    pltpu.sync_copy(deg_sh.at[pl.ds(sid * RPSD, RPSD)],
                    deg_out.at[cid, pl.ds(sid * RPSD, RPSD)])


@functools.partial(
    pl.kernel,
    out_type=jax.ShapeDtypeStruct((NC, NPA, D), jnp.float32),
    mesh=_sc_mesh,
    scratch_types=[
        pltpu.VMEM((NCHUNK, 1, CHUNK), jnp.int32),
        pltpu.VMEM((CHUNK,), jnp.int32),
        pltpu.VMEM((CHUNK,), jnp.int32),
        pltpu.VMEM((CHUNK, D), jnp.float32),
        pltpu.VMEM((CHUNK, D), jnp.float32),
        pltpu.VMEM_SHARED((NPA, D), jnp.float32),
        pltpu.SemaphoreType.DMA,
        pltpu.SemaphoreType.DMA,
    ],
)
def _msg_kernel(y_hbm, src_hbm, dst_hbm, zrows_hbm, acc_out,
                src_v, dst0, dst1, rows0, rows1, acc_sh, gsem0, gsem1):
    cid = lax.axis_index("c")
    sid = lax.axis_index("s")
    wid = sid * NC + cid
    pltpu.sync_copy(zrows_hbm, acc_sh.at[pl.ds(sid * RPSA, RPSA)])
    pltpu.sync_copy(src_hbm.at[wid], src_v)
    plsc.subcore_barrier()

    def fetch(j, dst_b, rows_b, sem):
        pltpu.async_copy(dst_hbm.at[wid, j, 0], dst_b, sem)
        pltpu.async_copy(y_hbm.at[src_v.at[j, 0]], rows_b, sem)

    def drain_and_add(j, dst_b, rows_b, sem):
        pltpu.make_async_copy(dst_hbm.at[wid, j, 0], dst_b, sem).wait()
        pltpu.make_async_copy(y_hbm.at[src_v.at[j, 0]], rows_b, sem).wait()
        pltpu.sync_copy(rows_b, acc_sh.at[dst_b], add=True)

    fetch(0, dst0, rows0, gsem0)

    def body(i, carry):
        j0 = 2 * i
        j1 = j0 + 1
        fetch(j1, dst1, rows1, gsem1)
        drain_and_add(j0, dst0, rows0, gsem0)
        fetch(j0 + 2, dst0, rows0, gsem0)
        drain_and_add(j1, dst1, rows1, gsem1)
        return carry

    lax.fori_loop(0, NPAIR, body, 0)
    drain_and_add(NCHUNK - 1, dst0, rows0, gsem0)
    plsc.subcore_barrier()
    pltpu.sync_copy(acc_sh.at[pl.ds(sid * RPSA, RPSA)],
                    acc_out.at[cid, pl.ds(sid * RPSA, RPSA)])


def _dinv(d0, d1):
    return lax.rsqrt(d0 + d1 + 1.0)


def _l1_body(x_ref, w1_ref, d0_ref, d1_ref, y_ref):
    dinv = _dinv(d0_ref[...], d1_ref[...])
    y_ref[...] = jnp.dot(x_ref[...], w1_ref[...],
                         preferred_element_type=jnp.float32) * dinv


def _l2_body(acc_ref0, acc_ref1, y1_ref, d0_ref, d1_ref, b1_ref, w2_ref,
             y2_ref):
    dinv = _dinv(d0_ref[...], d1_ref[...])
    h = dinv * (acc_ref0[0] + acc_ref1[0] + y1_ref[...]) + b1_ref[...]
    h = jnp.maximum(h, 0.0)
    y2_ref[...] = jnp.dot(h, w2_ref[...],
                          preferred_element_type=jnp.float32) * dinv


def _out_body(acc_ref0, acc_ref1, y2_ref, d0_ref, d1_ref, b2_ref, o_ref):
    dinv = _dinv(d0_ref[...], d1_ref[...])
    logits = dinv * (acc_ref0[0] + acc_ref1[0] + y2_ref[...]) + b2_ref[...]
    m = jnp.max(logits, axis=1, keepdims=True)
    lse = jnp.log(jnp.sum(jnp.exp(logits - m), axis=1, keepdims=True)) + m
    o_ref[...] = logits - lse


def _row_spec():
    return pl.BlockSpec((ROW_BLK, D), lambda i: (i, 0))


def _acc_spec(c):
    return pl.BlockSpec((1, ROW_BLK, D), lambda i: (c, i, 0))


def _deg_spec():
    return pl.BlockSpec((ROW_BLK, 1), lambda i: (i, 0))


def _full_spec():
    return pl.BlockSpec((D, D), lambda i: (0, 0))


def _bias_spec():
    return pl.BlockSpec((1, D), lambda i: (0, 0))


def kernel(x, edge_index, W1, b1, W2, b2):
    src = edge_index[0].astype(jnp.int32).reshape(NW, EPW)
    dst = edge_index[1].astype(jnp.int32).reshape(NW, EPW)
    src4 = jnp.pad(src, ((0, 0), (0, EPWP - EPW))).reshape(NW, NCHUNK, 1, CHUNK)
    dst4 = jnp.pad(dst, ((0, 0), (0, EPWP - EPW)),
                   constant_values=PAD_DST).reshape(NW, NCHUNK, 1, CHUNK)

    ones_c = jnp.ones((CHUNK,), jnp.float32)
    zeros_1d = jnp.zeros((RPSD,), jnp.float32)
    zeros_2d = jnp.zeros((RPSA, D), jnp.float32)

    degp = _deg_kernel(dst4, ones_c, zeros_1d)
    d0 = degp[0, :N_NODES].reshape(N_NODES, 1)
    d1 = degp[1, :N_NODES].reshape(N_NODES, 1)

    y1 = pl.pallas_call(
        _l1_body,
        grid=(N_BLKS,),
        in_specs=[_row_spec(), _full_spec(), _deg_spec(), _deg_spec()],
        out_specs=_row_spec(),
        out_shape=jax.ShapeDtypeStruct((N_NODES, D), jnp.float32),
    )(x, W1, d0, d1)

    accp1 = _msg_kernel(y1, src4, dst4, zeros_2d)

    y2 = pl.pallas_call(
        _l2_body,
        grid=(N_BLKS,),
        in_specs=[_acc_spec(0), _acc_spec(1), _row_spec(), _deg_spec(),
                  _deg_spec(), _bias_spec(), _full_spec()],
        out_specs=_row_spec(),
        out_shape=jax.ShapeDtypeStruct((N_NODES, D), jnp.float32),
    )(accp1, accp1, y1, d0, d1, b1.reshape(1, D), W2)

    accp2 = _msg_kernel(y2, src4, dst4, zeros_2d)

    out = pl.pallas_call(
        _out_body,
        grid=(N_BLKS,),
        in_specs=[_acc_spec(0), _acc_spec(1), _row_spec(), _deg_spec(),
                  _deg_spec(), _bias_spec()],
        out_specs=_row_spec(),
        out_shape=jax.ShapeDtypeStruct((N_NODES, D), jnp.float32),
    )(accp2, accp2, y2, d0, d1, b2.reshape(1, D))

    return out


# speedup vs baseline: 1.5174x; 1.5174x over previous
import functools

import jax
import jax.numpy as jnp
from jax import lax
from jax.experimental import pallas as pl
from jax.experimental.pallas import tpu as pltpu
from jax.experimental.pallas import tpu_sc as plsc

N_NODES = 10000
N_EDGES = 320000
D = 128

NC = 2
NS = 16
NW = NC * NS

EPW = N_EDGES // NW
CHUNK = 120
NCHUNK = 85
EPWP = NCHUNK * CHUNK
NPAIR = NCHUNK // 2
PAD_DST = N_NODES

NPD = 10240
RPSD = NPD // NS
NPA = 10112
RPSA = NPA // NS

DEG_AHEAD = 8

ROW_BLK = 1000
N_BLKS = N_NODES // ROW_BLK

_sc_mesh = plsc.VectorSubcoreMesh(core_axis_name="c", subcore_axis_name="s")


@functools.partial(
    pl.kernel,
    out_type=jax.ShapeDtypeStruct((NC, NPD), jnp.float32),
    mesh=_sc_mesh,
    scratch_types=[
        pltpu.VMEM((NCHUNK, 1, CHUNK), jnp.int32),
        pltpu.VMEM((CHUNK,), jnp.float32),
        pltpu.VMEM_SHARED((NPD,), jnp.float32),
        pltpu.SemaphoreType.DMA,
    ],
)
def _deg_kernel(dst_hbm, ones_hbm, zeros_hbm, deg_out, idx_v, ones_v, deg_sh,
                sem):
    cid = lax.axis_index("c")
    sid = lax.axis_index("s")
    wid = sid * NC + cid
    pltpu.sync_copy(zeros_hbm, deg_sh.at[pl.ds(sid * RPSD, RPSD)])
    pltpu.sync_copy(ones_hbm, ones_v)
    pltpu.sync_copy(dst_hbm.at[wid], idx_v)
    plsc.subcore_barrier()

    for i in range(DEG_AHEAD):
        pltpu.async_copy(ones_v, deg_sh.at[idx_v.at[i, 0]], sem, add=True)

    def body(i, carry):
        pltpu.make_async_copy(ones_v, deg_sh.at[idx_v.at[0, 0]], sem).wait()
        pltpu.async_copy(ones_v, deg_sh.at[idx_v.at[i + DEG_AHEAD, 0]], sem,
                         add=True)
        return carry

    lax.fori_loop(0, NCHUNK - DEG_AHEAD, body, 0)
    for _ in range(DEG_AHEAD):
        pltpu.make_async_copy(ones_v, deg_sh.at[idx_v.at[0, 0]], sem).wait()
    plsc.subcore_barrier()
    pltpu.sync_copy(deg_sh.at[pl.ds(sid * RPSD, RPSD)],
                    deg_out.at[cid, pl.ds(sid * RPSD, RPSD)])


@functools.partial(
    pl.kernel,
    out_type=jax.ShapeDtypeStruct((NC, NPA, D), jnp.float32),
    mesh=_sc_mesh,
    scratch_types=[
        pltpu.VMEM((NCHUNK, 1, CHUNK), jnp.int32),
        pltpu.VMEM((CHUNK,), jnp.int32),
        pltpu.VMEM((CHUNK,), jnp.int32),
        pltpu.VMEM((CHUNK, D), jnp.float32),
        pltpu.VMEM((CHUNK, D), jnp.float32),
        pltpu.VMEM_SHARED((NPA, D), jnp.float32),
        pltpu.SemaphoreType.DMA,
        pltpu.SemaphoreType.DMA,
    ],
)
def _msg_kernel(y_hbm, src_hbm, dst_hbm, zrows_hbm, acc_out,
                src_v, dst0, dst1, rows0, rows1, acc_sh, gsem0, gsem1):
    cid = lax.axis_index("c")
    sid = lax.axis_index("s")
    wid = sid * NC + cid
    pltpu.sync_copy(zrows_hbm, acc_sh.at[pl.ds(sid * RPSA, RPSA)])
    pltpu.sync_copy(src_hbm.at[wid], src_v)
    plsc.subcore_barrier()

    def fetch(j, dst_b, rows_b, sem):
        pltpu.async_copy(dst_hbm.at[wid, j, 0], dst_b, sem)
        pltpu.async_copy(y_hbm.at[src_v.at[j, 0]], rows_b, sem)

    def drain_and_add(j, dst_b, rows_b, sem):
        pltpu.make_async_copy(dst_hbm.at[wid, j, 0], dst_b, sem).wait()
        pltpu.make_async_copy(y_hbm.at[src_v.at[j, 0]], rows_b, sem).wait()
        pltpu.sync_copy(rows_b, acc_sh.at[dst_b], add=True)

    fetch(0, dst0, rows0, gsem0)

    def body(i, carry):
        j0 = 2 * i
        j1 = j0 + 1
        fetch(j1, dst1, rows1, gsem1)
        drain_and_add(j0, dst0, rows0, gsem0)
        fetch(j0 + 2, dst0, rows0, gsem0)
        drain_and_add(j1, dst1, rows1, gsem1)
        return carry

    lax.fori_loop(0, NPAIR, body, 0)
    drain_and_add(NCHUNK - 1, dst0, rows0, gsem0)
    plsc.subcore_barrier()
    pltpu.sync_copy(acc_sh.at[pl.ds(sid * RPSA, RPSA)],
                    acc_out.at[cid, pl.ds(sid * RPSA, RPSA)])


def _dinv(d0, d1):
    return lax.rsqrt(d0 + d1 + 1.0)


def _l1_body(x_ref, w1_ref, d0_ref, d1_ref, y_ref):
    dinv = _dinv(d0_ref[...], d1_ref[...])
    y_ref[...] = jnp.dot(x_ref[...], w1_ref[...],
                         preferred_element_type=jnp.float32) * dinv


def _l2_body(acc_ref0, acc_ref1, y1_ref, d0_ref, d1_ref, b1_ref, w2_ref,
             y2_ref):
    dinv = _dinv(d0_ref[...], d1_ref[...])
    h = dinv * (acc_ref0[0] + acc_ref1[0] + y1_ref[...]) + b1_ref[...]
    h = jnp.maximum(h, 0.0)
    y2_ref[...] = jnp.dot(h, w2_ref[...],
                          preferred_element_type=jnp.float32) * dinv


def _out_body(acc_ref0, acc_ref1, y2_ref, d0_ref, d1_ref, b2_ref, o_ref):
    dinv = _dinv(d0_ref[...], d1_ref[...])
    logits = dinv * (acc_ref0[0] + acc_ref1[0] + y2_ref[...]) + b2_ref[...]
    m = jnp.max(logits, axis=1, keepdims=True)
    lse = jnp.log(jnp.sum(jnp.exp(logits - m), axis=1, keepdims=True)) + m
    o_ref[...] = logits - lse


def _row_spec():
    return pl.BlockSpec((ROW_BLK, D), lambda i: (i, 0))


def _acc_spec(c):
    return pl.BlockSpec((1, ROW_BLK, D), lambda i: (c, i, 0))


def _deg_spec():
    return pl.BlockSpec((ROW_BLK, 1), lambda i: (i, 0))


def _full_spec():
    return pl.BlockSpec((D, D), lambda i: (0, 0))


def _bias_spec():
    return pl.BlockSpec((1, D), lambda i: (0, 0))


def kernel(x, edge_index, W1, b1, W2, b2):
    src = edge_index[0].astype(jnp.int32).reshape(NW, EPW)
    dst = edge_index[1].astype(jnp.int32).reshape(NW, EPW)
    src4 = jnp.pad(src, ((0, 0), (0, EPWP - EPW))).reshape(NW, NCHUNK, 1, CHUNK)
    dst4 = jnp.pad(dst, ((0, 0), (0, EPWP - EPW)),
                   constant_values=PAD_DST).reshape(NW, NCHUNK, 1, CHUNK)

    ones_c = jnp.ones((CHUNK,), jnp.float32)
    zeros_1d = jnp.zeros((RPSD,), jnp.float32)
    zeros_2d = jnp.zeros((RPSA, D), jnp.float32)

    degp = _deg_kernel(dst4, ones_c, zeros_1d)
    d0 = degp[0, :N_NODES].reshape(N_NODES, 1)
    d1 = degp[1, :N_NODES].reshape(N_NODES, 1)

    y1 = pl.pallas_call(
        _l1_body,
        grid=(N_BLKS,),
        in_specs=[_row_spec(), _full_spec(), _deg_spec(), _deg_spec()],
        out_specs=_row_spec(),
        out_shape=jax.ShapeDtypeStruct((N_NODES, D), jnp.float32),
    )(x, W1, d0, d1)

    accp1 = _msg_kernel(y1, src4, dst4, zeros_2d)

    y2 = pl.pallas_call(
        _l2_body,
        grid=(N_BLKS,),
        in_specs=[_acc_spec(0), _acc_spec(1), _row_spec(), _deg_spec(),
                  _deg_spec(), _bias_spec(), _full_spec()],
        out_specs=_row_spec(),
        out_shape=jax.ShapeDtypeStruct((N_NODES, D), jnp.float32),
    )(accp1, accp1, y1, d0, d1, b1.reshape(1, D), W2)

    accp2 = _msg_kernel(y2, src4, dst4, zeros_2d)

    out = pl.pallas_call(
        _out_body,
        grid=(N_BLKS,),
        in_specs=[_acc_spec(0), _acc_spec(1), _row_spec(), _deg_spec(),
                  _deg_spec(), _bias_spec()],
        out_specs=_row_spec(),
        out_shape=jax.ShapeDtypeStruct((N_NODES, D), jnp.float32),
    )(accp2, accp2, y2, d0, d1, b2.reshape(1, D))

    return out


# speedup vs baseline: 3.5866x; 2.3636x over previous
import functools

import jax
import jax.numpy as jnp
from jax import lax
from jax.experimental import pallas as pl
from jax.experimental.pallas import tpu as pltpu
from jax.experimental.pallas import tpu_sc as plsc

N_NODES = 10000
N_EDGES = 320000
D = 128

NC = 2
NS = 16
NW = NC * NS

EPW = N_EDGES // NW
CHUNK = 80
NCHUNK = 125
EPWP = NCHUNK * CHUNK
NPAIR = NCHUNK // 2
PAD_DST = N_NODES

NPD = 10240
RPSD = NPD // NS
NPA = 10240
RPSA = NPA // NS

DEG_AHEAD = 8

ROW_BLK = 1000
N_BLKS = N_NODES // ROW_BLK

_sc_mesh = plsc.VectorSubcoreMesh(core_axis_name="c", subcore_axis_name="s")


@functools.partial(
    pl.kernel,
    out_type=jax.ShapeDtypeStruct((NC, NPD), jnp.float32),
    mesh=_sc_mesh,
    scratch_types=[
        pltpu.VMEM((NCHUNK, 1, CHUNK), jnp.int32),
        pltpu.VMEM((CHUNK,), jnp.float32),
        pltpu.VMEM_SHARED((NPD,), jnp.float32),
        pltpu.SemaphoreType.DMA,
    ],
)
def _deg_kernel(dst_hbm, ones_hbm, zeros_hbm, deg_out, idx_v, ones_v, deg_sh,
                sem):
    cid = lax.axis_index("c")
    sid = lax.axis_index("s")
    wid = sid * NC + cid
    pltpu.sync_copy(zeros_hbm, deg_sh.at[pl.ds(sid * RPSD, RPSD)])
    pltpu.sync_copy(ones_hbm, ones_v)
    pltpu.sync_copy(dst_hbm.at[wid], idx_v)
    plsc.subcore_barrier()

    for i in range(DEG_AHEAD):
        pltpu.async_copy(ones_v, deg_sh.at[idx_v.at[i, 0]], sem, add=True)

    def body(i, carry):
        pltpu.make_async_copy(ones_v, deg_sh.at[idx_v.at[0, 0]], sem).wait()
        pltpu.async_copy(ones_v, deg_sh.at[idx_v.at[i + DEG_AHEAD, 0]], sem,
                         add=True)
        return carry

    lax.fori_loop(0, NCHUNK - DEG_AHEAD, body, 0)
    for _ in range(DEG_AHEAD):
        pltpu.make_async_copy(ones_v, deg_sh.at[idx_v.at[0, 0]], sem).wait()
    plsc.subcore_barrier()
    pltpu.sync_copy(deg_sh.at[pl.ds(sid * RPSD, RPSD)],
                    deg_out.at[cid, pl.ds(sid * RPSD, RPSD)])


@functools.partial(
    pl.kernel,
    out_type=jax.ShapeDtypeStruct((NC, NPA, D), jnp.float32),
    mesh=_sc_mesh,
    scratch_types=[
        pltpu.VMEM((NCHUNK, 1, CHUNK), jnp.int32),
        pltpu.VMEM((CHUNK,), jnp.int32),
        pltpu.VMEM((CHUNK,), jnp.int32),
        pltpu.VMEM((CHUNK, D), jnp.float32),
        pltpu.VMEM((CHUNK, D), jnp.float32),
        pltpu.VMEM_SHARED((NPA, D), jnp.float32),
        pltpu.SemaphoreType.DMA,
        pltpu.SemaphoreType.DMA,
    ],
)
def _msg_kernel(y_hbm, src_hbm, dst_hbm, zrows_hbm, acc_out,
                src_v, dst0, dst1, rows0, rows1, acc_sh, gsem0, gsem1):
    cid = lax.axis_index("c")
    sid = lax.axis_index("s")
    wid = sid * NC + cid
    pltpu.sync_copy(zrows_hbm, acc_sh.at[pl.ds(sid * RPSA, RPSA)])
    pltpu.sync_copy(src_hbm.at[wid], src_v)
    plsc.subcore_barrier()

    def fetch(j, dst_b, rows_b, sem):
        pltpu.async_copy(dst_hbm.at[wid, j, 0], dst_b, sem)
        pltpu.async_copy(y_hbm.at[src_v.at[j, 0]], rows_b, sem)

    def drain_and_add(j, dst_b, rows_b, sem):
        pltpu.make_async_copy(dst_hbm.at[wid, j, 0], dst_b, sem).wait()
        pltpu.make_async_copy(y_hbm.at[src_v.at[j, 0]], rows_b, sem).wait()
        pltpu.sync_copy(rows_b, acc_sh.at[dst_b], add=True)

    fetch(0, dst0, rows0, gsem0)

    def body(i, carry):
        j0 = 2 * i
        j1 = j0 + 1
        fetch(j1, dst1, rows1, gsem1)
        drain_and_add(j0, dst0, rows0, gsem0)
        fetch(j0 + 2, dst0, rows0, gsem0)
        drain_and_add(j1, dst1, rows1, gsem1)
        return carry

    lax.fori_loop(0, NPAIR, body, 0)
    drain_and_add(NCHUNK - 1, dst0, rows0, gsem0)
    plsc.subcore_barrier()
    pltpu.sync_copy(acc_sh.at[pl.ds(sid * RPSA, RPSA)],
                    acc_out.at[cid, pl.ds(sid * RPSA, RPSA)])


def _dinv(d0, d1):
    return lax.rsqrt(d0 + d1 + 1.0)


def _l1_body(x_ref, w1_ref, d0_ref, d1_ref, y_ref):
    dinv = _dinv(d0_ref[...], d1_ref[...])
    y_ref[...] = jnp.dot(x_ref[...], w1_ref[...],
                         preferred_element_type=jnp.float32) * dinv


def _l2_body(acc_ref0, acc_ref1, y1_ref, d0_ref, d1_ref, b1_ref, w2_ref,
             y2_ref):
    dinv = _dinv(d0_ref[...], d1_ref[...])
    h = dinv * (acc_ref0[0] + acc_ref1[0] + y1_ref[...]) + b1_ref[...]
    h = jnp.maximum(h, 0.0)
    y2_ref[...] = jnp.dot(h, w2_ref[...],
                          preferred_element_type=jnp.float32) * dinv


def _out_body(acc_ref0, acc_ref1, y2_ref, d0_ref, d1_ref, b2_ref, o_ref):
    dinv = _dinv(d0_ref[...], d1_ref[...])
    logits = dinv * (acc_ref0[0] + acc_ref1[0] + y2_ref[...]) + b2_ref[...]
    m = jnp.max(logits, axis=1, keepdims=True)
    lse = jnp.log(jnp.sum(jnp.exp(logits - m), axis=1, keepdims=True)) + m
    o_ref[...] = logits - lse


def _row_spec():
    return pl.BlockSpec((ROW_BLK, D), lambda i: (i, 0))


def _acc_spec(c):
    return pl.BlockSpec((1, ROW_BLK, D), lambda i: (c, i, 0))


def _deg_spec():
    return pl.BlockSpec((ROW_BLK, 1), lambda i: (i, 0))


def _full_spec():
    return pl.BlockSpec((D, D), lambda i: (0, 0))


def _bias_spec():
    return pl.BlockSpec((1, D), lambda i: (0, 0))


def kernel(x, edge_index, W1, b1, W2, b2):
    src = edge_index[0].astype(jnp.int32).reshape(NW, EPW)
    dst = edge_index[1].astype(jnp.int32).reshape(NW, EPW)
    src4 = jnp.pad(src, ((0, 0), (0, EPWP - EPW))).reshape(NW, NCHUNK, 1, CHUNK)
    dst4 = jnp.pad(dst, ((0, 0), (0, EPWP - EPW)),
                   constant_values=PAD_DST).reshape(NW, NCHUNK, 1, CHUNK)

    ones_c = jnp.ones((CHUNK,), jnp.float32)
    zeros_1d = jnp.zeros((RPSD,), jnp.float32)
    zeros_2d = jnp.zeros((RPSA, D), jnp.float32)

    degp = _deg_kernel(dst4, ones_c, zeros_1d)
    d0 = degp[0, :N_NODES].reshape(N_NODES, 1)
    d1 = degp[1, :N_NODES].reshape(N_NODES, 1)

    y1 = pl.pallas_call(
        _l1_body,
        grid=(N_BLKS,),
        in_specs=[_row_spec(), _full_spec(), _deg_spec(), _deg_spec()],
        out_specs=_row_spec(),
        out_shape=jax.ShapeDtypeStruct((N_NODES, D), jnp.float32),
    )(x, W1, d0, d1)

    accp1 = _msg_kernel(y1, src4, dst4, zeros_2d)

    y2 = pl.pallas_call(
        _l2_body,
        grid=(N_BLKS,),
        in_specs=[_acc_spec(0), _acc_spec(1), _row_spec(), _deg_spec(),
                  _deg_spec(), _bias_spec(), _full_spec()],
        out_specs=_row_spec(),
        out_shape=jax.ShapeDtypeStruct((N_NODES, D), jnp.float32),
    )(accp1, accp1, y1, d0, d1, b1.reshape(1, D), W2)

    accp2 = _msg_kernel(y2, src4, dst4, zeros_2d)

    out = pl.pallas_call(
        _out_body,
        grid=(N_BLKS,),
        in_specs=[_acc_spec(0), _acc_spec(1), _row_spec(), _deg_spec(),
                  _deg_spec(), _bias_spec()],
        out_specs=_row_spec(),
        out_shape=jax.ShapeDtypeStruct((N_NODES, D), jnp.float32),
    )(accp2, accp2, y2, d0, d1, b2.reshape(1, D))

    return out
